# baseline (speedup 1.0000x reference)
import functools

import jax
import jax.numpy as jnp
import numpy as np
from jax import lax
from jax.experimental import pallas as pl
from jax.experimental.pallas import tpu as pltpu
from jax.experimental.pallas import tpu_sc as plsc

K = 128
HIDDEN = 256
EDGE_TYPES = 1024

NUM_CORES = 2
NUM_SUBCORES = 16
NW = NUM_CORES * NUM_SUBCORES
CHUNK = 128


def _sc_gather_body(et_hbm, x_hbm, wedge_hbm, mul_hbm, bias_hbm,
                    rows_out, s_out,
                    idx0, idx1, rows0, rows1, rbf0, rbf1, x0, x1, s0, s1,
                    mul_v, bias_v, g0, g1, w0, w1):
    m_total = et_hbm.shape[0]
    per_w = m_total // NW
    n_pairs = per_w // (2 * CHUNK)
    wid = lax.axis_index("s") * NUM_CORES + lax.axis_index("c")
    wbase = wid * per_w

    pltpu.sync_copy(mul_hbm, mul_v)
    pltpu.sync_copy(bias_hbm, bias_v)

    def base_of(c):
        return pl.multiple_of(wbase + c * CHUNK, CHUNK)

    def compute_s(idx_v, x_v, s_v):
        for j in range(CHUNK // 16):
            sl = pl.ds(j * 16, 16)
            idx16 = idx_v[sl]
            m16 = plsc.load_gather(mul_v, [idx16])
            b16 = plsc.load_gather(bias_v, [idx16])
            z = m16 * x_v[sl] + b16
            s_v[sl] = 1.0 / (1.0 + jnp.exp(-z))

    def pack_rows(rows_v, rbf_v):
        def row_body(r8, carry):
            for u in range(8):
                r = r8 * 8 + u
                for t in range(K // 32):
                    a = rows_v[r, pl.ds(32 * t, 16)]
                    b = rows_v[r, pl.ds(32 * t + 16, 16)]
                    packed = plsc.pack(
                        a, b, format=plsc.PackFormat.INTERLEAVED)
                    rbf_v[pl.ds(r * (K // 2) + 16 * t, 16)] = plsc.bitcast(
                        packed, jnp.int32)
            return carry
        lax.fori_loop(0, CHUNK // 8, row_body, 0)

    bp = base_of(0)
    pltpu.sync_copy(et_hbm.at[pl.ds(bp, CHUNK)], idx0)
    pltpu.async_copy(wedge_hbm.at[idx0], rows0, g0)

    def pair_body(o, carry):
        c0 = 2 * o
        b0 = base_of(c0)
        b1 = base_of(c0 + 1)
        pltpu.sync_copy(et_hbm.at[pl.ds(b1, CHUNK)], idx1)
        pltpu.async_copy(wedge_hbm.at[idx1], rows1, g1)
        pltpu.sync_copy(x_hbm.at[pl.ds(b0, CHUNK)], x0)
        compute_s(idx0, x0, s0)
        pltpu.make_async_copy(wedge_hbm.at[idx0], rows0, g0).wait()
        pack_rows(rows0, rbf0)
        pltpu.async_copy(rbf0, rows_out.at[pl.ds(b0 * (K // 2), CHUNK * (K // 2))], w0)
        pltpu.sync_copy(s0, s_out.at[pl.ds(b0, CHUNK)])
        pltpu.sync_copy(x_hbm.at[pl.ds(b1, CHUNK)], x1)
        compute_s(idx1, x1, s1)
        pltpu.make_async_copy(wedge_hbm.at[idx1], rows1, g1).wait()
        pack_rows(rows1, rbf1)
        pltpu.async_copy(rbf1, rows_out.at[pl.ds(b1 * (K // 2), CHUNK * (K // 2))], w1)
        pltpu.sync_copy(s1, s_out.at[pl.ds(b1, CHUNK)])
        @pl.when(o < n_pairs - 1)
        def _():
            b2 = base_of(c0 + 2)
            pltpu.sync_copy(et_hbm.at[pl.ds(b2, CHUNK)], idx0)
            pltpu.async_copy(wedge_hbm.at[idx0], rows0, g0)

        pltpu.make_async_copy(rbf0, rows_out.at[pl.ds(b0 * (K // 2), CHUNK * (K // 2))], w0).wait()
        pltpu.make_async_copy(rbf1, rows_out.at[pl.ds(b1 * (K // 2), CHUNK * (K // 2))], w1).wait()
        return carry

    lax.fori_loop(0, n_pairs, pair_body, 0)


def _make_sc_gather(m_total):
    mesh = plsc.VectorSubcoreMesh(core_axis_name="c", subcore_axis_name="s")
    return pl.kernel(
        _sc_gather_body,
        out_type=[
            jax.ShapeDtypeStruct((m_total * (K // 2),), jnp.int32),
            jax.ShapeDtypeStruct((m_total,), jnp.float32),
        ],
        mesh=mesh,
        scratch_types=[
            pltpu.VMEM((CHUNK,), jnp.int32),
            pltpu.VMEM((CHUNK,), jnp.int32),
            pltpu.VMEM((CHUNK, K), jnp.float32),
            pltpu.VMEM((CHUNK, K), jnp.float32),
            pltpu.VMEM((CHUNK * (K // 2),), jnp.int32),
            pltpu.VMEM((CHUNK * (K // 2),), jnp.int32),
            pltpu.VMEM((CHUNK,), jnp.float32),
            pltpu.VMEM((CHUNK,), jnp.float32),
            pltpu.VMEM((CHUNK,), jnp.float32),
            pltpu.VMEM((CHUNK,), jnp.float32),
            pltpu.VMEM((EDGE_TYPES,), jnp.float32),
            pltpu.VMEM((EDGE_TYPES,), jnp.float32),
            pltpu.SemaphoreType.DMA,
            pltpu.SemaphoreType.DMA,
            pltpu.SemaphoreType.DMA,
            pltpu.SemaphoreType.DMA,
        ],
        compiler_params=pltpu.CompilerParams(needs_layout_passes=False),
    )


def _tc_mlp_body(x_ref, s_ref, rows_ref, l1w_ref, l1b_ref, l2w_ref,
                 l2b_ref, lng_ref, lnb_ref, o_ref):
    t = rows_ref.shape[0]
    x = x_ref[...].reshape(t, 1)
    s = s_ref[...].reshape(t, 1)
    h1 = x * l1w_ref[...] + l1b_ref[...]
    g = 0.5 * h1 * (1.0 + lax.erf(h1 * 0.7071067811865476))
    h2 = jnp.dot(g.astype(jnp.bfloat16), l2w_ref[...],
                 preferred_element_type=jnp.float32)
    h2 = h2 + l2b_ref[...]
    mean = jnp.mean(h2, axis=-1, keepdims=True)
    m2 = jnp.mean(h2 * h2, axis=-1, keepdims=True)
    var = m2 - mean * mean
    hn = (h2 - mean) * lax.rsqrt(var + 1e-5) * lng_ref[...] + lnb_ref[...]
    o_ref[...] = hn + rows_ref[...].astype(jnp.float32) * s


def _tc_mlp_chunk(xc, sc, rows, l1w, l1b, l2wt, l2b, lng, lnb,
                  m_total, chunk_idx, block_t, big=None):
    m_chunk = xc.shape[0]
    n_blocks = m_chunk // block_t
    blk0 = chunk_idx * n_blocks
    body = _tc_mlp_body if big is None else (
        lambda b_ref, *refs: _tc_mlp_body(*refs))
    in_specs = [
        pl.BlockSpec((block_t,), lambda i: (i,)),
        pl.BlockSpec((block_t,), lambda i: (i,)),
        pl.BlockSpec((block_t, K), lambda i: (i, 0)),
        pl.BlockSpec((1, HIDDEN), lambda i: (0, 0)),
        pl.BlockSpec((1, HIDDEN), lambda i: (0, 0)),
        pl.BlockSpec((HIDDEN, K), lambda i: (0, 0)),
        pl.BlockSpec((1, K), lambda i: (0, 0)),
        pl.BlockSpec((1, K), lambda i: (0, 0)),
        pl.BlockSpec((1, K), lambda i: (0, 0)),
    ]
    args = [xc, sc, rows, l1w, l1b, l2wt, l2b, lng, lnb]
    kwargs = {}
    if big is not None:
        in_specs = [pl.BlockSpec(memory_space=pltpu.MemorySpace.HBM)] + in_specs
        args = [big] + args
        kwargs["input_output_aliases"] = {0: 0}
    return pl.pallas_call(
        body,
        grid=(n_blocks,),
        in_specs=in_specs,
        out_specs=pl.BlockSpec((block_t, K), lambda i: (blk0 + i, 0)),
        out_shape=jax.ShapeDtypeStruct((m_total, K), jnp.float32),
        compiler_params=pltpu.CompilerParams(
            dimension_semantics=("arbitrary",),
        ),
        **kwargs,
    )(*args)


def kernel(x, edge_type, mul_w, bias_w, w_edge_w, l1_w, l1_b, l2_w, l2_b,
           ln_g, ln_b):
    orig_shape = x.shape
    m_total = x.size
    xf = x.reshape(m_total)
    et = edge_type.reshape(m_total).astype(jnp.int32)

    mul_flat = mul_w.reshape(EDGE_TYPES)
    bias_flat = bias_w.reshape(EDGE_TYPES)
    l1w = l1_w.reshape(1, HIDDEN)
    l1b = l1_b.reshape(1, HIDDEN)
    l2wt = l2_w.T.astype(jnp.bfloat16)
    l2b = l2_b.reshape(1, K)
    lng = ln_g.reshape(1, K)
    lnb = ln_b.reshape(1, K)

    perm = np.arange(K).reshape(K // 32, 16, 2).transpose(0, 2, 1).reshape(K)
    wedge_perm = w_edge_w[:, perm]

    n_chunks = 4
    m_chunk = m_total // n_chunks
    sc_gather = _make_sc_gather(m_chunk)
    sc_outs = []
    for c in range(n_chunks):
        sl = slice(c * m_chunk, (c + 1) * m_chunk)
        sc_outs.append(sc_gather(et[sl], xf[sl], wedge_perm,
                                 mul_flat, bias_flat))

    big = None
    for c in range(n_chunks):
        rows, s = sc_outs[c]
        rows = jax.lax.bitcast_convert_type(
            rows, jnp.bfloat16).reshape(m_chunk, K)
        sl = slice(c * m_chunk, (c + 1) * m_chunk)
        big = _tc_mlp_chunk(xf[sl], s, rows, l1w, l1b, l2wt, l2b, lng, lnb,
                            m_total, c, block_t=2048, big=big)
    return big.reshape(orig_shape + (K,))


# speedup vs baseline: 3.2983x; 3.2983x over previous
import jax
import jax.numpy as jnp
import numpy as np
from jax import lax
from jax.experimental import pallas as pl
from jax.experimental.pallas import tpu as pltpu
from jax.experimental.pallas import tpu_sc as plsc

K = 128
HIDDEN = 256
EDGE_TYPES = 1024

NUM_CORES = 2
NUM_SUBCORES = 16
NW = NUM_CORES * NUM_SUBCORES
CHUNK = 128


def _sc_gather_body(et_hbm, x_hbm, wedge_hbm, mul_hbm, bias_hbm,
                    rows_out, s_out,
                    idx0, idx1, rows0, rows1, x0, x1, s0, s1,
                    mul_v, bias_v, g0, g1, w0, w1):
    m_total = et_hbm.shape[0]
    per_w = m_total // NW
    n_pairs = per_w // (2 * CHUNK)
    wid = lax.axis_index("s") * NUM_CORES + lax.axis_index("c")
    wbase = wid * per_w

    pltpu.sync_copy(mul_hbm, mul_v)
    pltpu.sync_copy(bias_hbm, bias_v)

    def base_of(c):
        return pl.multiple_of(wbase + c * CHUNK, CHUNK)

    def compute_s(idx_v, x_v, s_v):
        for j in range(CHUNK // 16):
            sl = pl.ds(j * 16, 16)
            idx16 = idx_v[sl]
            m16 = plsc.load_gather(mul_v, [idx16])
            b16 = plsc.load_gather(bias_v, [idx16])
            z = m16 * x_v[sl] + b16
            s_v[sl] = 1.0 / (1.0 + jnp.exp(-z))

    bp = base_of(0)
    pltpu.sync_copy(et_hbm.at[pl.ds(bp, CHUNK)], idx0)
    pltpu.async_copy(wedge_hbm.at[idx0], rows0, g0)

    def pair_body(o, carry):
        c0 = 2 * o
        b0 = base_of(c0)
        b1 = base_of(c0 + 1)
        pltpu.sync_copy(et_hbm.at[pl.ds(b1, CHUNK)], idx1)
        pltpu.async_copy(wedge_hbm.at[idx1], rows1, g1)
        pltpu.sync_copy(x_hbm.at[pl.ds(b0, CHUNK)], x0)
        compute_s(idx0, x0, s0)
        pltpu.make_async_copy(wedge_hbm.at[idx0], rows0, g0).wait()
        pltpu.async_copy(rows0, rows_out.at[pl.ds(b0, CHUNK)], w0)
        pltpu.sync_copy(s0, s_out.at[pl.ds(b0, CHUNK)])
        pltpu.sync_copy(x_hbm.at[pl.ds(b1, CHUNK)], x1)
        compute_s(idx1, x1, s1)
        pltpu.make_async_copy(wedge_hbm.at[idx1], rows1, g1).wait()
        pltpu.async_copy(rows1, rows_out.at[pl.ds(b1, CHUNK)], w1)
        pltpu.sync_copy(s1, s_out.at[pl.ds(b1, CHUNK)])
        pltpu.make_async_copy(rows0, rows_out.at[pl.ds(b0, CHUNK)], w0).wait()

        @pl.when(o < n_pairs - 1)
        def _():
            b2 = base_of(c0 + 2)
            pltpu.sync_copy(et_hbm.at[pl.ds(b2, CHUNK)], idx0)
            pltpu.async_copy(wedge_hbm.at[idx0], rows0, g0)

        pltpu.make_async_copy(rows1, rows_out.at[pl.ds(b1, CHUNK)], w1).wait()
        return carry

    lax.fori_loop(0, n_pairs, pair_body, 0)


def _make_sc_gather(m_total):
    mesh = plsc.VectorSubcoreMesh(core_axis_name="c", subcore_axis_name="s")
    return pl.kernel(
        _sc_gather_body,
        out_type=[
            jax.ShapeDtypeStruct((m_total, K), jnp.float32),
            jax.ShapeDtypeStruct((m_total,), jnp.float32),
        ],
        mesh=mesh,
        scratch_types=[
            pltpu.VMEM((CHUNK,), jnp.int32),
            pltpu.VMEM((CHUNK,), jnp.int32),
            pltpu.VMEM((CHUNK, K), jnp.float32),
            pltpu.VMEM((CHUNK, K), jnp.float32),
            pltpu.VMEM((CHUNK,), jnp.float32),
            pltpu.VMEM((CHUNK,), jnp.float32),
            pltpu.VMEM((CHUNK,), jnp.float32),
            pltpu.VMEM((CHUNK,), jnp.float32),
            pltpu.VMEM((EDGE_TYPES,), jnp.float32),
            pltpu.VMEM((EDGE_TYPES,), jnp.float32),
            pltpu.SemaphoreType.DMA,
            pltpu.SemaphoreType.DMA,
            pltpu.SemaphoreType.DMA,
            pltpu.SemaphoreType.DMA,
        ],
        compiler_params=pltpu.CompilerParams(needs_layout_passes=False),
    )


def _tc_mlp_body(x_ref, s_ref, rows_ref, l1w_ref, l1b_ref, l2w_ref,
                 l2b_ref, lng_ref, lnb_ref, o_ref):
    t = rows_ref.shape[0]
    x = x_ref[...].reshape(t, 1)
    s = s_ref[...].reshape(t, 1)
    h1 = x * l1w_ref[...] + l1b_ref[...]
    g = 0.5 * h1 * (1.0 + lax.erf(h1 * 0.7071067811865476))
    h2 = jnp.dot(g.astype(jnp.bfloat16), l2w_ref[...],
                 preferred_element_type=jnp.float32)
    h2 = h2 + l2b_ref[...]
    mean = jnp.mean(h2, axis=-1, keepdims=True)
    m2 = jnp.mean(h2 * h2, axis=-1, keepdims=True)
    var = m2 - mean * mean
    hn = (h2 - mean) * lax.rsqrt(var + 1e-5) * lng_ref[...] + lnb_ref[...]
    o_ref[...] = hn + rows_ref[...] * s


def _tc_mlp_chunk(xc, sc, rows, l1w, l1b, l2wt, l2b, lng, lnb,
                  m_total, chunk_idx, block_t, big=None):
    m_chunk = xc.shape[0]
    n_blocks = m_chunk // block_t
    blk0 = chunk_idx * n_blocks
    body = _tc_mlp_body if big is None else (
        lambda b_ref, *refs: _tc_mlp_body(*refs))
    in_specs = [
        pl.BlockSpec((block_t,), lambda i: (i,)),
        pl.BlockSpec((block_t,), lambda i: (i,)),
        pl.BlockSpec((block_t, K), lambda i: (i, 0)),
        pl.BlockSpec((1, HIDDEN), lambda i: (0, 0)),
        pl.BlockSpec((1, HIDDEN), lambda i: (0, 0)),
        pl.BlockSpec((HIDDEN, K), lambda i: (0, 0)),
        pl.BlockSpec((1, K), lambda i: (0, 0)),
        pl.BlockSpec((1, K), lambda i: (0, 0)),
        pl.BlockSpec((1, K), lambda i: (0, 0)),
    ]
    args = [xc, sc, rows, l1w, l1b, l2wt, l2b, lng, lnb]
    kwargs = {}
    if big is not None:
        in_specs = [pl.BlockSpec(memory_space=pltpu.MemorySpace.HBM)] + in_specs
        args = [big] + args
        kwargs["input_output_aliases"] = {0: 0}
    return pl.pallas_call(
        body,
        grid=(n_blocks,),
        in_specs=in_specs,
        out_specs=pl.BlockSpec((block_t, K), lambda i: (blk0 + i, 0)),
        out_shape=jax.ShapeDtypeStruct((m_total, K), jnp.float32),
        compiler_params=pltpu.CompilerParams(
            dimension_semantics=("arbitrary",),
        ),
        **kwargs,
    )(*args)


def kernel(x, edge_type, mul_w, bias_w, w_edge_w, l1_w, l1_b, l2_w, l2_b,
           ln_g, ln_b):
    orig_shape = x.shape
    m_total = x.size
    xf = x.reshape(m_total)
    et = edge_type.reshape(m_total).astype(jnp.int32)

    mul_flat = mul_w.reshape(EDGE_TYPES)
    bias_flat = bias_w.reshape(EDGE_TYPES)
    l1w = l1_w.reshape(1, HIDDEN)
    l1b = l1_b.reshape(1, HIDDEN)
    l2wt = l2_w.T.astype(jnp.bfloat16)
    l2b = l2_b.reshape(1, K)
    lng = ln_g.reshape(1, K)
    lnb = ln_b.reshape(1, K)

    n_chunks = 4
    m_chunk = m_total // n_chunks
    sc_gather = _make_sc_gather(m_chunk)
    sc_outs = []
    for c in range(n_chunks):
        sl = slice(c * m_chunk, (c + 1) * m_chunk)
        sc_outs.append(sc_gather(et[sl], xf[sl], w_edge_w,
                                 mul_flat, bias_flat))

    big = None
    for c in range(n_chunks):
        rows, s = sc_outs[c]
        sl = slice(c * m_chunk, (c + 1) * m_chunk)
        big = _tc_mlp_chunk(xf[sl], s, rows, l1w, l1b, l2wt, l2b, lng, lnb,
                            m_total, c, block_t=2048, big=big)
    return big.reshape(orig_shape + (K,))


# speedup vs baseline: 3.3168x; 1.0056x over previous
import jax
import jax.numpy as jnp
import numpy as np
from jax import lax
from jax.experimental import pallas as pl
from jax.experimental.pallas import tpu as pltpu
from jax.experimental.pallas import tpu_sc as plsc

K = 128
HIDDEN = 256
EDGE_TYPES = 1024

NUM_CORES = 2
NUM_SUBCORES = 16
NW = NUM_CORES * NUM_SUBCORES
CHUNK = 128


def _sc_gather_body(et_hbm, x_hbm, wedge_hbm, mul_hbm, bias_hbm,
                    rows_out, s_out,
                    idx0, idx1, rows0, rows1, x0, x1, s0, s1,
                    mul_v, bias_v, g0, g1, w0, w1):
    m_total = et_hbm.shape[0]
    per_w = m_total // NW
    n_pairs = per_w // (2 * CHUNK)
    wid = lax.axis_index("s") * NUM_CORES + lax.axis_index("c")
    wbase = wid * per_w

    pltpu.sync_copy(mul_hbm, mul_v)
    pltpu.sync_copy(bias_hbm, bias_v)

    def base_of(c):
        return pl.multiple_of(wbase + c * CHUNK, CHUNK)

    def compute_s(idx_v, x_v, s_v):
        for j in range(CHUNK // 16):
            sl = pl.ds(j * 16, 16)
            idx16 = idx_v[sl]
            m16 = plsc.load_gather(mul_v, [idx16])
            b16 = plsc.load_gather(bias_v, [idx16])
            z = m16 * x_v[sl] + b16
            s_v[sl] = 1.0 / (1.0 + jnp.exp(-z))

    bp = base_of(0)
    pltpu.sync_copy(et_hbm.at[pl.ds(bp, CHUNK)], idx0)
    pltpu.async_copy(wedge_hbm.at[idx0], rows0, g0)

    def pair_body(o, carry):
        c0 = 2 * o
        b0 = base_of(c0)
        b1 = base_of(c0 + 1)
        pltpu.sync_copy(et_hbm.at[pl.ds(b1, CHUNK)], idx1)
        pltpu.async_copy(wedge_hbm.at[idx1], rows1, g1)
        pltpu.sync_copy(x_hbm.at[pl.ds(b0, CHUNK)], x0)
        compute_s(idx0, x0, s0)
        pltpu.make_async_copy(wedge_hbm.at[idx0], rows0, g0).wait()
        pltpu.async_copy(rows0, rows_out.at[pl.ds(b0, CHUNK)], w0)
        pltpu.sync_copy(s0, s_out.at[pl.ds(b0, CHUNK)])
        pltpu.sync_copy(x_hbm.at[pl.ds(b1, CHUNK)], x1)
        compute_s(idx1, x1, s1)
        pltpu.make_async_copy(wedge_hbm.at[idx1], rows1, g1).wait()
        pltpu.async_copy(rows1, rows_out.at[pl.ds(b1, CHUNK)], w1)
        pltpu.sync_copy(s1, s_out.at[pl.ds(b1, CHUNK)])
        pltpu.make_async_copy(rows0, rows_out.at[pl.ds(b0, CHUNK)], w0).wait()

        @pl.when(o < n_pairs - 1)
        def _():
            b2 = base_of(c0 + 2)
            pltpu.sync_copy(et_hbm.at[pl.ds(b2, CHUNK)], idx0)
            pltpu.async_copy(wedge_hbm.at[idx0], rows0, g0)

        pltpu.make_async_copy(rows1, rows_out.at[pl.ds(b1, CHUNK)], w1).wait()
        return carry

    lax.fori_loop(0, n_pairs, pair_body, 0)


def _make_sc_gather(m_total):
    mesh = plsc.VectorSubcoreMesh(core_axis_name="c", subcore_axis_name="s")
    return pl.kernel(
        _sc_gather_body,
        out_type=[
            jax.ShapeDtypeStruct((m_total, K), jnp.float32),
            jax.ShapeDtypeStruct((m_total,), jnp.float32),
        ],
        mesh=mesh,
        scratch_types=[
            pltpu.VMEM((CHUNK,), jnp.int32),
            pltpu.VMEM((CHUNK,), jnp.int32),
            pltpu.VMEM((CHUNK, K), jnp.float32),
            pltpu.VMEM((CHUNK, K), jnp.float32),
            pltpu.VMEM((CHUNK,), jnp.float32),
            pltpu.VMEM((CHUNK,), jnp.float32),
            pltpu.VMEM((CHUNK,), jnp.float32),
            pltpu.VMEM((CHUNK,), jnp.float32),
            pltpu.VMEM((EDGE_TYPES,), jnp.float32),
            pltpu.VMEM((EDGE_TYPES,), jnp.float32),
            pltpu.SemaphoreType.DMA,
            pltpu.SemaphoreType.DMA,
            pltpu.SemaphoreType.DMA,
            pltpu.SemaphoreType.DMA,
        ],
        compiler_params=pltpu.CompilerParams(needs_layout_passes=False),
    )


def _tc_mlp_body(x_ref, s_ref, rows_ref, l1w_ref, l1b_ref, l2w_ref,
                 l2b_ref, lng_ref, lnb_ref, o_ref):
    t = rows_ref.shape[0]
    x = x_ref[...].reshape(t, 1)
    s = s_ref[...].reshape(t, 1)
    h1 = x * l1w_ref[...] + l1b_ref[...]
    g = 0.5 * h1 * (1.0 + lax.erf(h1 * 0.7071067811865476))
    h2 = jnp.dot(g.astype(jnp.bfloat16), l2w_ref[...],
                 preferred_element_type=jnp.float32)
    h2 = h2 + l2b_ref[...]
    mean = jnp.mean(h2, axis=-1, keepdims=True)
    m2 = jnp.mean(h2 * h2, axis=-1, keepdims=True)
    var = m2 - mean * mean
    hn = (h2 - mean) * lax.rsqrt(var + 1e-5) * lng_ref[...] + lnb_ref[...]
    o_ref[...] = hn + rows_ref[...] * s


def _tc_mlp_chunk(xc, sc, rows, l1w, l1b, l2wt, l2b, lng, lnb,
                  m_total, chunk_idx, block_t, big=None):
    m_chunk = xc.shape[0]
    n_blocks = m_chunk // block_t
    blk0 = chunk_idx * n_blocks
    body = _tc_mlp_body if big is None else (
        lambda b_ref, *refs: _tc_mlp_body(*refs))
    in_specs = [
        pl.BlockSpec((block_t,), lambda i: (i,)),
        pl.BlockSpec((block_t,), lambda i: (i,)),
        pl.BlockSpec((block_t, K), lambda i: (i, 0)),
        pl.BlockSpec((1, HIDDEN), lambda i: (0, 0)),
        pl.BlockSpec((1, HIDDEN), lambda i: (0, 0)),
        pl.BlockSpec((HIDDEN, K), lambda i: (0, 0)),
        pl.BlockSpec((1, K), lambda i: (0, 0)),
        pl.BlockSpec((1, K), lambda i: (0, 0)),
        pl.BlockSpec((1, K), lambda i: (0, 0)),
    ]
    args = [xc, sc, rows, l1w, l1b, l2wt, l2b, lng, lnb]
    kwargs = {}
    if big is not None:
        in_specs = [pl.BlockSpec(memory_space=pltpu.MemorySpace.HBM)] + in_specs
        args = [big] + args
        kwargs["input_output_aliases"] = {0: 0}
    return pl.pallas_call(
        body,
        grid=(n_blocks,),
        in_specs=in_specs,
        out_specs=pl.BlockSpec((block_t, K), lambda i: (blk0 + i, 0)),
        out_shape=jax.ShapeDtypeStruct((m_total, K), jnp.float32),
        compiler_params=pltpu.CompilerParams(
            dimension_semantics=("arbitrary",),
        ),
        **kwargs,
    )(*args)


def kernel(x, edge_type, mul_w, bias_w, w_edge_w, l1_w, l1_b, l2_w, l2_b,
           ln_g, ln_b):
    orig_shape = x.shape
    m_total = x.size
    xf = x.reshape(m_total)
    et = edge_type.reshape(m_total).astype(jnp.int32)

    mul_flat = mul_w.reshape(EDGE_TYPES)
    bias_flat = bias_w.reshape(EDGE_TYPES)
    l1w = l1_w.reshape(1, HIDDEN)
    l1b = l1_b.reshape(1, HIDDEN)
    l2wt = l2_w.T.astype(jnp.bfloat16)
    l2b = l2_b.reshape(1, K)
    lng = ln_g.reshape(1, K)
    lnb = ln_b.reshape(1, K)

    n_chunks = 8
    m_chunk = m_total // n_chunks
    sc_gather = _make_sc_gather(m_chunk)
    sc_outs = []
    for c in range(n_chunks):
        sl = slice(c * m_chunk, (c + 1) * m_chunk)
        sc_outs.append(sc_gather(et[sl], xf[sl], w_edge_w,
                                 mul_flat, bias_flat))

    big = None
    for c in range(n_chunks):
        rows, s = sc_outs[c]
        sl = slice(c * m_chunk, (c + 1) * m_chunk)
        big = _tc_mlp_chunk(xf[sl], s, rows, l1w, l1b, l2wt, l2b, lng, lnb,
                            m_total, c, block_t=2048, big=big)
    return big.reshape(orig_shape + (K,))
